# Initial kernel scaffold; baseline (speedup 1.0000x reference)
import functools

import jax
import jax.numpy as jnp
from jax import lax
from jax.experimental import pallas as pl
from jax.experimental.pallas import tpu as pltpu
from jax.experimental.pallas import tpu_sc as plsc

NC = 2
NS = 16
NW = NC * NS
LANES = 16


def _matmul_kernel(x_ref, w_ref, o_ref):
    o_ref[...] = jnp.dot(x_ref[...], w_ref[...],
                         preferred_element_type=jnp.float32)


def _matmul(x, w):
    n, d_in = x.shape
    d_out = w.shape[1]
    blk = 2000
    grid = (n // blk,)
    return pl.pallas_call(
        _matmul_kernel,
        grid=grid,
        in_specs=[
            pl.BlockSpec((blk, d_in), lambda i: (i, 0)),
            pl.BlockSpec((d_in, d_out), lambda i: (0, 0)),
        ],
        out_specs=pl.BlockSpec((blk, d_out), lambda i: (i, 0)),
        out_shape=jax.ShapeDtypeStruct((n, d_out), jnp.float32),
    )(x, w)


def _combine_kernel(p_ref, w_ref, o_ref):
    o_ref[...] = (p_ref[0] + p_ref[1]) * w_ref[0, 0]


def _combine(partials, attn_w):
    _, n, d = partials.shape
    blk = 2000
    grid = (n // blk,)
    return pl.pallas_call(
        _combine_kernel,
        grid=grid,
        in_specs=[
            pl.BlockSpec((NC, blk, d), lambda i: (0, i, 0)),
            pl.BlockSpec(memory_space=pltpu.SMEM),
        ],
        out_specs=pl.BlockSpec((blk, d), lambda i: (i, 0)),
        out_shape=jax.ShapeDtypeStruct((n, d), jnp.float32),
    )(partials, attn_w)


def _sc_edge(xp, src, dst):
    n, dm = xp.shape
    _, n_chunks, bsz = src.shape
    rows_per_tile = n // NS
    zr = 125
    ksl = dm // LANES

    mesh = plsc.VectorSubcoreMesh(core_axis_name="c", subcore_axis_name="s")

    @functools.partial(
        pl.kernel,
        mesh=mesh,
        out_type=jax.ShapeDtypeStruct((NC, n, dm), jnp.float32),
        scratch_types=[
            pltpu.VMEM((n_chunks, bsz), jnp.int32),
            pltpu.VMEM((n_chunks, bsz), jnp.int32),
            pltpu.VMEM((bsz, dm), jnp.float32),
            pltpu.VMEM((bsz, dm), jnp.float32),
            pltpu.VMEM((zr, dm), jnp.float32),
            pltpu.VMEM_SHARED((n, dm), jnp.float32),
            pltpu.SemaphoreType.DMA,
            pltpu.SemaphoreType.DMA,
        ],
    )
    def edge_k(xp_hbm, src_hbm, dst_hbm, out_hbm,
               sidx_v, didx_v, srows_v, drows_v, zbuf_v, acc_sh, sem0, sem1):
        cid = lax.axis_index("c")
        sid = lax.axis_index("s")
        wid = cid * NS + sid
        tbase = sid * rows_per_tile

        def zrow(r, carry):
            for k in range(ksl):
                zbuf_v[r, pl.ds(k * LANES, LANES)] = jnp.zeros(
                    (LANES,), jnp.float32)
            return carry
        lax.fori_loop(0, zr, zrow, 0)
        for i in range(rows_per_tile // zr):
            pltpu.sync_copy(zbuf_v, acc_sh.at[pl.ds(tbase + i * zr, zr)])

        pltpu.sync_copy(src_hbm.at[wid], sidx_v)
        pltpu.sync_copy(dst_hbm.at[wid], didx_v)
        plsc.subcore_barrier()

        def chunk(j, carry):
            cp_s = pltpu.async_copy(xp_hbm.at[sidx_v.at[j]], srows_v, sem0)
            cp_d = pltpu.async_copy(xp_hbm.at[didx_v.at[j]], drows_v, sem1)
            cp_s.wait()
            cp_d.wait()

            def edge(e, ecarry):
                svecs = []
                p = jnp.zeros((LANES,), jnp.float32)
                for k in range(ksl):
                    s = srows_v[e, pl.ds(k * LANES, LANES)]
                    d = drows_v[e, pl.ds(k * LANES, LANES)]
                    svecs.append(s)
                    p = p + s * d
                dot = jnp.sum(p)
                for k in range(ksl):
                    srows_v[e, pl.ds(k * LANES, LANES)] = svecs[k] * dot
                return ecarry
            lax.fori_loop(0, bsz, edge, 0)

            pltpu.sync_copy(srows_v, acc_sh.at[didx_v.at[j]], add=True)
            return carry
        lax.fori_loop(0, n_chunks, chunk, 0)

        plsc.subcore_barrier()
        pltpu.sync_copy(acc_sh.at[pl.ds(tbase, rows_per_tile)],
                        out_hbm.at[cid, pl.ds(tbase, rows_per_tile)])

    return edge_k(xp, src, dst)


def kernel(X, edge_index, weights, attention_w):
    n = X.shape[0]
    e = edge_index.shape[1]
    e_per_w = e // NW
    bsz = 80
    n_chunks = e_per_w // bsz

    xp = _matmul(X, weights)
    src = edge_index[0].reshape(NW, n_chunks, bsz)
    dst = edge_index[1].reshape(NW, n_chunks, bsz)
    partials = _sc_edge(xp, src, dst)
    return _combine(partials, attention_w)


# speedup vs baseline: 3.5009x; 3.5009x over previous
import functools

import jax
import jax.numpy as jnp
from jax import lax
from jax.experimental import pallas as pl
from jax.experimental.pallas import tpu as pltpu
from jax.experimental.pallas import tpu_sc as plsc

NC = 2
NS = 16
NW = NC * NS
LANES = 16

_GDN = lax.GatherDimensionNumbers(
    offset_dims=(), collapsed_slice_dims=(0,), start_index_map=(0,))


def _shuffle(v, idx):
    return lax.gather(v, idx[:, None], _GDN, (1,),
                      mode=lax.GatherScatterMode.PROMISE_IN_BOUNDS)


def _matmul_kernel(x_ref, w_ref, o_ref):
    o_ref[...] = jnp.dot(x_ref[...], w_ref[...],
                         preferred_element_type=jnp.float32)


def _matmul(x, w):
    n, d_in = x.shape
    d_out = w.shape[1]
    blk = 2048
    grid = (n // blk,)
    return pl.pallas_call(
        _matmul_kernel,
        grid=grid,
        in_specs=[
            pl.BlockSpec((blk, d_in), lambda i: (i, 0)),
            pl.BlockSpec((d_in, d_out), lambda i: (0, 0)),
        ],
        out_specs=pl.BlockSpec((blk, d_out), lambda i: (i, 0)),
        out_shape=jax.ShapeDtypeStruct((n, d_out), jnp.float32),
    )(x, w)


def _sc_combine(partials, aw16):
    _, n_pad, dm = partials.shape
    rows_per_w = n_pad // NW
    cb = 80
    ksl = dm // LANES
    mesh = plsc.VectorSubcoreMesh(core_axis_name="c", subcore_axis_name="s")

    @functools.partial(
        pl.kernel,
        mesh=mesh,
        out_type=jax.ShapeDtypeStruct((n_pad, dm), jnp.float32),
        scratch_types=[
            pltpu.VMEM((cb, dm), jnp.float32),
            pltpu.VMEM((cb, dm), jnp.float32),
            pltpu.VMEM((LANES,), jnp.float32),
        ],
    )
    def k2(p_hbm, aw_hbm, o_hbm, b0_v, b1_v, aw_v):
        cid = lax.axis_index("c")
        sid = lax.axis_index("s")
        wid = cid * NS + sid
        wbase = wid * rows_per_w
        pltpu.sync_copy(aw_hbm, aw_v)
        wv = aw_v[pl.ds(0, LANES)]

        def blk(i, carry):
            rbase = pl.multiple_of(wbase + i * cb, 8)
            pltpu.sync_copy(p_hbm.at[0, pl.ds(rbase, cb)], b0_v)
            pltpu.sync_copy(p_hbm.at[1, pl.ds(rbase, cb)], b1_v)

            def row(r, rcarry):
                for k in range(ksl):
                    sl = pl.ds(k * LANES, LANES)
                    b0_v[r, sl] = (b0_v[r, sl] + b1_v[r, sl]) * wv
                return rcarry
            lax.fori_loop(0, cb, row, 0)
            pltpu.sync_copy(b0_v, o_hbm.at[pl.ds(rbase, cb)])
            return carry
        lax.fori_loop(0, rows_per_w // cb, blk, 0)

    return k2(partials, aw16)


def _sc_edge(xp, src, dst, n_chunks, bsz):
    n_pad, dm = xp.shape
    e_per_w = n_chunks * bsz
    rows_per_tile = n_pad // NS
    ksl = dm // LANES
    ngrp = bsz // LANES

    mesh = plsc.VectorSubcoreMesh(core_axis_name="c", subcore_axis_name="s")

    @functools.partial(
        pl.kernel,
        mesh=mesh,
        out_type=jax.ShapeDtypeStruct((NC, n_pad, dm), jnp.float32),
        scratch_types=[
            pltpu.VMEM((bsz,), jnp.int32),
            pltpu.VMEM((bsz,), jnp.int32),
            pltpu.VMEM((bsz, dm), jnp.float32),
            pltpu.VMEM((bsz, dm), jnp.float32),
            pltpu.VMEM_SHARED((n_pad, dm), jnp.float32),
            pltpu.SemaphoreType.DMA,
            pltpu.SemaphoreType.DMA,
        ],
    )
    def k1(xp_hbm, src_hbm, dst_hbm, acc_hbm,
           sidx_v, didx_v, srows_v, drows_v, acc_sh, sem0, sem1):
        cid = lax.axis_index("c")
        sid = lax.axis_index("s")
        wid = cid * NS + sid
        tbase = sid * rows_per_tile
        lane = lax.iota(jnp.int32, LANES)

        def zrow(r, carry):
            for k in range(ksl):
                srows_v[r, pl.ds(k * LANES, LANES)] = jnp.zeros(
                    (LANES,), jnp.float32)
            return carry
        lax.fori_loop(0, bsz, zrow, 0)
        for i in range(rows_per_tile // bsz):
            pltpu.sync_copy(srows_v, acc_sh.at[pl.ds(tbase + i * bsz, bsz)])
        plsc.subcore_barrier()

        def chunk(j, carry):
            ebase = wid * e_per_w + j * bsz
            pltpu.sync_copy(src_hbm.at[pl.ds(ebase, bsz)], sidx_v)
            pltpu.sync_copy(dst_hbm.at[pl.ds(ebase, bsz)], didx_v)
            cp_s = pltpu.async_copy(xp_hbm.at[sidx_v], srows_v, sem0)
            cp_d = pltpu.async_copy(xp_hbm.at[didx_v], drows_v, sem1)
            cp_s.wait()
            cp_d.wait()

            def grp(g, gcarry):
                base = g * LANES
                for i in range(LANES):
                    e = base + i
                    svecs = []
                    p = jnp.zeros((LANES,), jnp.float32)
                    for k in range(ksl):
                        s = srows_v[e, pl.ds(k * LANES, LANES)]
                        d = drows_v[e, pl.ds(k * LANES, LANES)]
                        svecs.append(s)
                        p = p + s * d
                    for sh in (8, 4, 2, 1):
                        p = p + _shuffle(p, jnp.bitwise_xor(lane, sh))
                    for k in range(ksl):
                        srows_v[e, pl.ds(k * LANES, LANES)] = svecs[k] * p
                return gcarry
            lax.fori_loop(0, ngrp, grp, 0)

            pltpu.sync_copy(srows_v, acc_sh.at[didx_v], add=True)
            return carry
        lax.fori_loop(0, n_chunks, chunk, 0)

        plsc.subcore_barrier()
        pltpu.sync_copy(acc_sh.at[pl.ds(tbase, rows_per_tile)],
                        acc_hbm.at[cid, pl.ds(tbase, rows_per_tile)])

    return k1(xp, src, dst)


def kernel(X, edge_index, weights, attention_w):
    n, d_in = X.shape
    e = edge_index.shape[1]
    e_per_w = e // NW
    bsz = 80
    n_chunks = e_per_w // bsz
    n_pad = ((n + 2559) // 2560) * 2560

    x_pad = jnp.pad(X, ((0, n_pad - n), (0, 0)))
    xp = _matmul(x_pad, weights)

    partials = _sc_edge(xp, edge_index[0], edge_index[1], n_chunks, bsz)
    aw16 = jnp.full((LANES,), attention_w[0, 0], jnp.float32)
    return _sc_combine(partials, aw16)[:n]


# speedup vs baseline: 7.0071x; 2.0015x over previous
import functools

import jax
import jax.numpy as jnp
from jax import lax
from jax.experimental import pallas as pl
from jax.experimental.pallas import tpu as pltpu
from jax.experimental.pallas import tpu_sc as plsc

NC = 2
NS = 16
NW = NC * NS
LANES = 16

_GDN = lax.GatherDimensionNumbers(
    offset_dims=(), collapsed_slice_dims=(0,), start_index_map=(0,))


def _shuffle(v, idx):
    return lax.gather(v, idx[:, None], _GDN, (1,),
                      mode=lax.GatherScatterMode.PROMISE_IN_BOUNDS)


def _matmul_kernel(x_ref, w_ref, o_ref):
    o_ref[...] = jnp.dot(x_ref[...], w_ref[...],
                         preferred_element_type=jnp.float32)


def _matmul(x, w):
    n, d_in = x.shape
    d_out = w.shape[1]
    blk = 2048
    grid = (n // blk,)
    return pl.pallas_call(
        _matmul_kernel,
        grid=grid,
        in_specs=[
            pl.BlockSpec((blk, d_in), lambda i: (i, 0)),
            pl.BlockSpec((d_in, d_out), lambda i: (0, 0)),
        ],
        out_specs=pl.BlockSpec((blk, d_out), lambda i: (i, 0)),
        out_shape=jax.ShapeDtypeStruct((n, d_out), jnp.float32),
    )(x, w)


def _sc_combine(partials, aw16):
    _, n_pad, dm = partials.shape
    rows_per_w = n_pad // NW
    cb = 80
    ksl = dm // LANES
    mesh = plsc.VectorSubcoreMesh(core_axis_name="c", subcore_axis_name="s")

    @functools.partial(
        pl.kernel,
        mesh=mesh,
        out_type=jax.ShapeDtypeStruct((n_pad, dm), jnp.float32),
        scratch_types=[
            pltpu.VMEM((cb, dm), jnp.float32),
            pltpu.VMEM((cb, dm), jnp.float32),
            pltpu.VMEM((LANES,), jnp.float32),
        ],
    )
    def k2(p_hbm, aw_hbm, o_hbm, b0_v, b1_v, aw_v):
        cid = lax.axis_index("c")
        sid = lax.axis_index("s")
        wid = cid * NS + sid
        wbase = wid * rows_per_w
        pltpu.sync_copy(aw_hbm, aw_v)
        wv = aw_v[pl.ds(0, LANES)]

        def blk(i, carry):
            rbase = pl.multiple_of(wbase + i * cb, 8)
            pltpu.sync_copy(p_hbm.at[0, pl.ds(rbase, cb)], b0_v)
            pltpu.sync_copy(p_hbm.at[1, pl.ds(rbase, cb)], b1_v)

            def row(r, rcarry):
                for k in range(ksl):
                    sl = pl.ds(k * LANES, LANES)
                    b0_v[r, sl] = (b0_v[r, sl] + b1_v[r, sl]) * wv
                return rcarry
            lax.fori_loop(0, cb, row, 0)
            pltpu.sync_copy(b0_v, o_hbm.at[pl.ds(rbase, cb)])
            return carry
        lax.fori_loop(0, rows_per_w // cb, blk, 0)

    return k2(partials, aw16)


def _sc_edge(xp, src, dst, n_chunks, bsz):
    n_pad, dm = xp.shape
    e_per_w = n_chunks * bsz
    rows_per_tile = n_pad // NS
    ksl = dm // LANES
    ngrp = bsz // LANES

    mesh = plsc.VectorSubcoreMesh(core_axis_name="c", subcore_axis_name="s")

    assert n_chunks % 2 == 1

    @functools.partial(
        pl.kernel,
        mesh=mesh,
        out_type=jax.ShapeDtypeStruct((NC, n_pad, dm), jnp.float32),
        scratch_types=[
            pltpu.VMEM((bsz,), jnp.int32),
            pltpu.VMEM((bsz,), jnp.int32),
            pltpu.VMEM((bsz,), jnp.int32),
            pltpu.VMEM((bsz,), jnp.int32),
            pltpu.VMEM((bsz, dm), jnp.float32),
            pltpu.VMEM((bsz, dm), jnp.float32),
            pltpu.VMEM((bsz, dm), jnp.float32),
            pltpu.VMEM((bsz, dm), jnp.float32),
            pltpu.VMEM_SHARED((n_pad, dm), jnp.float32),
            pltpu.SemaphoreType.DMA,
            pltpu.SemaphoreType.DMA,
            pltpu.SemaphoreType.DMA,
            pltpu.SemaphoreType.DMA,
            pltpu.SemaphoreType.DMA,
            pltpu.SemaphoreType.DMA,
            pltpu.SemaphoreType.DMA,
            pltpu.SemaphoreType.DMA,
        ],
    )
    def k1(xp_hbm, src_hbm, dst_hbm, acc_hbm,
           sidx_v0, sidx_v1, didx_v0, didx_v1,
           srows_v0, srows_v1, drows_v0, drows_v1, acc_sh,
           sgs0, sgs1, sgd0, sgd1, sw0, sw1, si, sj):
        cid = lax.axis_index("c")
        sid = lax.axis_index("s")
        wid = cid * NS + sid
        tbase = sid * rows_per_tile
        lane = lax.iota(jnp.int32, LANES)
        sidx = (sidx_v0, sidx_v1)
        didx = (didx_v0, didx_v1)
        srows = (srows_v0, srows_v1)
        drows = (drows_v0, drows_v1)
        sgs = (sgs0, sgs1)
        sgd = (sgd0, sgd1)
        sw = (sw0, sw1)

        def zrow(r, carry):
            for k in range(ksl):
                srows_v0[r, pl.ds(k * LANES, LANES)] = jnp.zeros(
                    (LANES,), jnp.float32)
            return carry
        lax.fori_loop(0, bsz, zrow, 0)
        for i in range(rows_per_tile // bsz):
            pltpu.sync_copy(srows_v0, acc_sh.at[pl.ds(tbase + i * bsz, bsz)])
        plsc.subcore_barrier()

        def start_idx(j, b):
            ebase = wid * e_per_w + j * bsz
            pltpu.make_async_copy(
                src_hbm.at[pl.ds(ebase, bsz)], sidx[b], si).start()
            pltpu.make_async_copy(
                dst_hbm.at[pl.ds(ebase, bsz)], didx[b], sj).start()

        def wait_idx(b):
            pltpu.make_async_copy(
                src_hbm.at[pl.ds(0, bsz)], sidx[b], si).wait()
            pltpu.make_async_copy(
                dst_hbm.at[pl.ds(0, bsz)], didx[b], sj).wait()

        def start_gather(b):
            pltpu.make_async_copy(xp_hbm.at[sidx[b]], srows[b], sgs[b]).start()
            pltpu.make_async_copy(xp_hbm.at[didx[b]], drows[b], sgd[b]).start()

        def wait_gather(b):
            pltpu.make_async_copy(xp_hbm.at[sidx[b]], srows[b], sgs[b]).wait()
            pltpu.make_async_copy(xp_hbm.at[didx[b]], drows[b], sgd[b]).wait()

        def start_scatter(b):
            pltpu.make_async_copy(
                srows[b], acc_sh.at[didx[b]], sw[b]).start(add=True)

        def wait_scatter(b):
            pltpu.make_async_copy(srows[b], acc_sh.at[didx[b]], sw[b]).wait()

        def compute(b):
            sr = srows[b]
            dr = drows[b]

            def edge(e, ecarry):
                svecs = []
                p = jnp.zeros((LANES,), jnp.float32)
                for k in range(ksl):
                    s = sr[e, pl.ds(k * LANES, LANES)]
                    d = dr[e, pl.ds(k * LANES, LANES)]
                    svecs.append(s)
                    p = p + s * d
                for sh in (8, 4, 2, 1):
                    p = p + _shuffle(p, jnp.bitwise_xor(lane, sh))
                for k in range(ksl):
                    sr[e, pl.ds(k * LANES, LANES)] = svecs[k] * p
                return ecarry
            lax.fori_loop(0, bsz, edge, 0)

        start_idx(0, 0)
        wait_idx(0)
        start_gather(0)

        def step(j, b, first):
            wait_gather(b)
            if not first:
                wait_scatter(1 - b)
            start_idx(j + 1, 1 - b)
            wait_idx(1 - b)
            start_gather(1 - b)
            compute(b)
            start_scatter(b)

        def pair(jj, carry):
            j = jj * 2
            step(j, 0, False)
            step(j + 1, 1, False)
            return carry

        step(0, 0, True)
        step(1, 1, False)
        lax.fori_loop(1, (n_chunks - 1) // 2, pair, 0)

        wait_gather(0)
        wait_scatter(1)
        compute(0)
        start_scatter(0)
        wait_scatter(0)

        plsc.subcore_barrier()
        pltpu.sync_copy(acc_sh.at[pl.ds(tbase, rows_per_tile)],
                        acc_hbm.at[cid, pl.ds(tbase, rows_per_tile)])

    return k1(xp, src, dst)


def kernel(X, edge_index, weights, attention_w):
    n, d_in = X.shape
    e = edge_index.shape[1]
    e_per_w = e // NW
    bsz = 80
    n_chunks = e_per_w // bsz
    n_pad = ((n + 2559) // 2560) * 2560

    x_pad = jnp.pad(X, ((0, n_pad - n), (0, 0)))
    xp = _matmul(x_pad, weights)

    partials = _sc_edge(xp, edge_index[0], edge_index[1], n_chunks, bsz)
    aw16 = jnp.full((LANES,), attention_w[0, 0], jnp.float32)
    return _sc_combine(partials, aw16)[:n]


# speedup vs baseline: 9.0165x; 1.2868x over previous
import functools

import jax
import jax.numpy as jnp
from jax import lax
from jax.experimental import pallas as pl
from jax.experimental.pallas import tpu as pltpu
from jax.experimental.pallas import tpu_sc as plsc

NC = 2
NS = 16
NW = NC * NS
LANES = 16

_GDN = lax.GatherDimensionNumbers(
    offset_dims=(), collapsed_slice_dims=(0,), start_index_map=(0,))


def _shuffle(v, idx):
    return lax.gather(v, idx[:, None], _GDN, (1,),
                      mode=lax.GatherScatterMode.PROMISE_IN_BOUNDS)


def _matmul_kernel(x_ref, w_ref, o_ref):
    o_ref[...] = jnp.dot(x_ref[...], w_ref[...],
                         preferred_element_type=jnp.float32)


def _matmul(x, w):
    n, d_in = x.shape
    d_out = w.shape[1]
    blk = 2048
    grid = (n // blk,)
    return pl.pallas_call(
        _matmul_kernel,
        grid=grid,
        in_specs=[
            pl.BlockSpec((blk, d_in), lambda i: (i, 0)),
            pl.BlockSpec((d_in, d_out), lambda i: (0, 0)),
        ],
        out_specs=pl.BlockSpec((blk, d_out), lambda i: (i, 0)),
        out_shape=jax.ShapeDtypeStruct((n, d_out), jnp.float32),
    )(x, w)


def _sc_combine(partials, aw16):
    _, n_pad, dm = partials.shape
    rows_per_w = n_pad // NW
    cb = 80
    ksl = dm // LANES
    mesh = plsc.VectorSubcoreMesh(core_axis_name="c", subcore_axis_name="s")

    @functools.partial(
        pl.kernel,
        mesh=mesh,
        out_type=jax.ShapeDtypeStruct((n_pad, dm), jnp.float32),
        scratch_types=[
            pltpu.VMEM((cb, dm), jnp.float32),
            pltpu.VMEM((cb, dm), jnp.float32),
            pltpu.VMEM((LANES,), jnp.float32),
        ],
    )
    def k2(p_hbm, aw_hbm, o_hbm, b0_v, b1_v, aw_v):
        cid = lax.axis_index("c")
        sid = lax.axis_index("s")
        wid = cid * NS + sid
        wbase = wid * rows_per_w
        pltpu.sync_copy(aw_hbm, aw_v)
        wv = aw_v[pl.ds(0, LANES)]

        def blk(i, carry):
            rbase = pl.multiple_of(wbase + i * cb, 8)
            pltpu.sync_copy(p_hbm.at[0, pl.ds(rbase, cb)], b0_v)
            pltpu.sync_copy(p_hbm.at[1, pl.ds(rbase, cb)], b1_v)

            def row(r, rcarry):
                for k in range(ksl):
                    sl = pl.ds(k * LANES, LANES)
                    b0_v[r, sl] = (b0_v[r, sl] + b1_v[r, sl]) * wv
                return rcarry
            lax.fori_loop(0, cb, row, 0)
            pltpu.sync_copy(b0_v, o_hbm.at[pl.ds(rbase, cb)])
            return carry
        lax.fori_loop(0, rows_per_w // cb, blk, 0)

    return k2(partials, aw16)


def _sc_edge(xp, src, dst, n_chunks, bsz):
    n_pad, dm = xp.shape
    e_per_w = n_chunks * bsz
    rows_per_tile = n_pad // NS
    ksl = dm // LANES
    ngrp = bsz // LANES

    mesh = plsc.VectorSubcoreMesh(core_axis_name="c", subcore_axis_name="s")

    assert n_chunks % 2 == 1

    @functools.partial(
        pl.kernel,
        mesh=mesh,
        out_type=jax.ShapeDtypeStruct((NC, n_pad, dm), jnp.float32),
        scratch_types=[
            pltpu.VMEM((bsz,), jnp.int32),
            pltpu.VMEM((bsz,), jnp.int32),
            pltpu.VMEM((bsz,), jnp.int32),
            pltpu.VMEM((bsz,), jnp.int32),
            pltpu.VMEM((bsz, dm), jnp.float32),
            pltpu.VMEM((bsz, dm), jnp.float32),
            pltpu.VMEM((bsz, dm), jnp.float32),
            pltpu.VMEM((bsz, dm), jnp.float32),
            pltpu.VMEM_SHARED((n_pad, dm), jnp.float32),
            pltpu.SemaphoreType.DMA,
            pltpu.SemaphoreType.DMA,
            pltpu.SemaphoreType.DMA,
            pltpu.SemaphoreType.DMA,
            pltpu.SemaphoreType.DMA,
            pltpu.SemaphoreType.DMA,
            pltpu.SemaphoreType.DMA,
            pltpu.SemaphoreType.DMA,
        ],
    )
    def k1(xp_hbm, src_hbm, dst_hbm, acc_hbm,
           sidx_v0, sidx_v1, didx_v0, didx_v1,
           srows_v0, srows_v1, drows_v0, drows_v1, acc_sh,
           sgs0, sgs1, sgd0, sgd1, sw0, sw1, si, sj):
        cid = lax.axis_index("c")
        sid = lax.axis_index("s")
        wid = cid * NS + sid
        tbase = sid * rows_per_tile
        lane = lax.iota(jnp.int32, LANES)
        sidx = (sidx_v0, sidx_v1)
        didx = (didx_v0, didx_v1)
        srows = (srows_v0, srows_v1)
        drows = (drows_v0, drows_v1)
        sgs = (sgs0, sgs1)
        sgd = (sgd0, sgd1)
        sw = (sw0, sw1)

        def zrow(r, carry):
            for k in range(ksl):
                srows_v0[r, pl.ds(k * LANES, LANES)] = jnp.zeros(
                    (LANES,), jnp.float32)
            return carry
        lax.fori_loop(0, bsz, zrow, 0)
        for i in range(rows_per_tile // bsz):
            pltpu.sync_copy(srows_v0, acc_sh.at[pl.ds(tbase + i * bsz, bsz)])
        plsc.subcore_barrier()

        def start_idx(j, b):
            ebase = wid * e_per_w + j * bsz
            pltpu.make_async_copy(
                src_hbm.at[pl.ds(ebase, bsz)], sidx[b], si).start()
            pltpu.make_async_copy(
                dst_hbm.at[pl.ds(ebase, bsz)], didx[b], sj).start()

        def wait_idx(b):
            pltpu.make_async_copy(
                src_hbm.at[pl.ds(0, bsz)], sidx[b], si).wait()
            pltpu.make_async_copy(
                dst_hbm.at[pl.ds(0, bsz)], didx[b], sj).wait()

        def start_gather(b):
            pltpu.make_async_copy(xp_hbm.at[sidx[b]], srows[b], sgs[b]).start()
            pltpu.make_async_copy(xp_hbm.at[didx[b]], drows[b], sgd[b]).start()

        def wait_gather(b):
            pltpu.make_async_copy(xp_hbm.at[sidx[b]], srows[b], sgs[b]).wait()
            pltpu.make_async_copy(xp_hbm.at[didx[b]], drows[b], sgd[b]).wait()

        def start_scatter(b):
            pltpu.make_async_copy(
                srows[b], acc_sh.at[didx[b]], sw[b]).start(add=True)

        def wait_scatter(b):
            pltpu.make_async_copy(srows[b], acc_sh.at[didx[b]], sw[b]).wait()

        def compute(b):
            sr = srows[b]
            dr = drows[b]

            @plsc.parallel_loop(0, bsz, 1, unroll=2)
            def edge(e):
                svecs = []
                p = jnp.zeros((LANES,), jnp.float32)
                for k in range(ksl):
                    s = sr[e, pl.ds(k * LANES, LANES)]
                    d = dr[e, pl.ds(k * LANES, LANES)]
                    svecs.append(s)
                    p = p + s * d
                for sh in (8, 4, 2, 1):
                    p = p + _shuffle(p, jnp.bitwise_xor(lane, sh))
                for k in range(ksl):
                    sr[e, pl.ds(k * LANES, LANES)] = svecs[k] * p

        start_idx(0, 0)
        wait_idx(0)
        start_gather(0)

        def step(j, b, first):
            wait_gather(b)
            if not first:
                wait_scatter(1 - b)
            start_idx(j + 1, 1 - b)
            wait_idx(1 - b)
            start_gather(1 - b)
            compute(b)
            start_scatter(b)

        def pair(jj, carry):
            j = jj * 2
            step(j, 0, False)
            step(j + 1, 1, False)
            return carry

        step(0, 0, True)
        step(1, 1, False)
        lax.fori_loop(1, (n_chunks - 1) // 2, pair, 0)

        wait_gather(0)
        wait_scatter(1)
        compute(0)
        start_scatter(0)
        wait_scatter(0)

        plsc.subcore_barrier()
        pltpu.sync_copy(acc_sh.at[pl.ds(tbase, rows_per_tile)],
                        acc_hbm.at[cid, pl.ds(tbase, rows_per_tile)])

    return k1(xp, src, dst)


def kernel(X, edge_index, weights, attention_w):
    n, d_in = X.shape
    e = edge_index.shape[1]
    e_per_w = e // NW
    bsz = 80
    n_chunks = e_per_w // bsz
    n_pad = ((n + 2559) // 2560) * 2560

    x_pad = jnp.pad(X, ((0, n_pad - n), (0, 0)))
    xp = _matmul(x_pad, weights)

    partials = _sc_edge(xp, edge_index[0], edge_index[1], n_chunks, bsz)
    aw16 = jnp.full((LANES,), attention_w[0, 0], jnp.float32)
    return _sc_combine(partials, aw16)[:n]
